# baseline (speedup 1.0000x reference)
import functools

import jax
import jax.numpy as jnp
from jax import lax
from jax.experimental import pallas as pl
from jax.experimental.pallas import tpu as pltpu
from jax.experimental.pallas import tpu_sc as plsc

D_MODEL = 1024
LANES = 16
NUM_CORES = 2
NUM_SUBCORES = 16
NUM_WORKERS = NUM_CORES * NUM_SUBCORES
B_TOTAL = 32768
B_PER_W = B_TOTAL // NUM_WORKERS
CHUNK = 16
NX = 4
NP = 2
N_CHUNKS = B_PER_W // CHUNK
N_SUPER = N_CHUNKS // NX
N_SLICES = CHUNK * (D_MODEL // LANES)


def _sc_body(x_hbm, pos_hbm, pe_hbm, out_hbm,
             idx_all, pe_v, x_v, gat_sem, xin_sem, out_sem):
    wid = lax.axis_index("s") * NUM_CORES + lax.axis_index("c")
    base = wid * B_PER_W

    pltpu.sync_copy(pos_hbm.at[pl.ds(base, B_PER_W)], idx_all)

    def idx_ref(c):
        return idx_all.at[pl.ds(c * CHUNK, CHUNK)]

    def rows(c):
        return pl.ds(base + c * CHUNK, CHUNK)

    def issue_loads(c, sx, sp):
        pltpu.async_copy(pe_hbm.at[idx_ref(c)], pe_v.at[sp], gat_sem.at[sp])
        pltpu.async_copy(x_hbm.at[rows(c)], x_v.at[sx], xin_sem.at[sx])

    issue_loads(0, 0, 0)

    def super_step(g, _):
        for b in range(NX):
            c = g * NX + b
            sx, sp = b, b % NP
            cl = c + 1
            sx_l, sp_l = (b + 1) % NX, (b + 1) % NP

            @pl.when(cl < N_CHUNKS)
            def _():
                @pl.when(cl >= NX)
                def _():
                    pltpu.make_async_copy(
                        x_v.at[sx_l], out_hbm.at[rows(cl - NX)],
                        out_sem.at[sx_l]).wait()
                issue_loads(cl, sx_l, sp_l)

            pltpu.make_async_copy(
                pe_hbm.at[idx_ref(c)], pe_v.at[sp], gat_sem.at[sp]).wait()
            pltpu.make_async_copy(
                x_hbm.at[rows(c)], x_v.at[sx], xin_sem.at[sx]).wait()

            n_sl = D_MODEL // LANES

            @plsc.parallel_loop(0, N_SLICES, 1, unroll=8)
            def _(i):
                t = i // n_sl
                j = i - t * n_sl
                d = pl.ds(j * LANES, LANES)
                plsc.addupdate(x_v.at[sx, t, d], pe_v[sp, t, d])

            pltpu.async_copy(x_v.at[sx], out_hbm.at[rows(c)], out_sem.at[sx])
        return 0

    lax.fori_loop(0, N_SUPER, super_step, 0)

    for b in range(NX):
        c = N_CHUNKS - NX + b
        pltpu.make_async_copy(
            x_v.at[b], out_hbm.at[rows(c)], out_sem.at[b]).wait()


@jax.jit
def _pe_add(x2d, pos1d, pe):
    mesh = plsc.VectorSubcoreMesh(core_axis_name="c", subcore_axis_name="s")
    kern = functools.partial(
        pl.kernel,
        mesh=mesh,
        out_type=jax.ShapeDtypeStruct((B_TOTAL, D_MODEL), jnp.float32),
        scratch_types=[
            pltpu.VMEM((B_PER_W,), jnp.int32),
            pltpu.VMEM((NP, CHUNK, D_MODEL), jnp.float32),
            pltpu.VMEM((NX, CHUNK, D_MODEL), jnp.float32),
            pltpu.SemaphoreType.DMA((NP,)),
            pltpu.SemaphoreType.DMA((NX,)),
            pltpu.SemaphoreType.DMA((NX,)),
        ],
    )(_sc_body)
    return kern(x2d, pos1d, pe)


def kernel(x, position, pe):
    b, s, d = x.shape
    x2d = x.reshape(b * s, d)
    pos1d = position.reshape(b * s).astype(jnp.int32)
    out = _pe_add(x2d, pos1d, pe)
    return out.reshape(b, s, d)


# speedup vs baseline: 1.0168x; 1.0168x over previous
import functools

import jax
import jax.numpy as jnp
from jax import lax
from jax.experimental import pallas as pl
from jax.experimental.pallas import tpu as pltpu
from jax.experimental.pallas import tpu_sc as plsc

D_MODEL = 1024
LANES = 16
NUM_CORES = 2
NUM_SUBCORES = 16
NUM_WORKERS = NUM_CORES * NUM_SUBCORES
B_TOTAL = 32768
B_PER_W = B_TOTAL // NUM_WORKERS
CHUNK = 8
NX = 8
NP = 4
LOOKAHEAD = 3
N_CHUNKS = B_PER_W // CHUNK
N_SUPER = N_CHUNKS // NX
N_SLICES = CHUNK * (D_MODEL // LANES)


def _sc_body(x_hbm, pos_hbm, pe_hbm, out_hbm,
             idx_all, pe_v, x_v, gat_sem, xin_sem, out_sem):
    wid = lax.axis_index("s") * NUM_CORES + lax.axis_index("c")
    base = wid * B_PER_W

    pltpu.sync_copy(pos_hbm.at[pl.ds(base, B_PER_W)], idx_all)

    def idx_ref(c):
        return idx_all.at[pl.ds(c * CHUNK, CHUNK)]

    def rows(c):
        return pl.ds(base + c * CHUNK, CHUNK)

    def issue_loads(c, sx, sp):
        pltpu.async_copy(pe_hbm.at[idx_ref(c)], pe_v.at[sp], gat_sem.at[sp])
        pltpu.async_copy(x_hbm.at[rows(c)], x_v.at[sx], xin_sem.at[sx])

    for c in range(LOOKAHEAD):
        issue_loads(c, c, c)

    def super_step(g, _):
        for b in range(NX):
            c = g * NX + b
            sx, sp = b, b % NP
            cl = c + LOOKAHEAD
            sx_l, sp_l = (b + LOOKAHEAD) % NX, (b + LOOKAHEAD) % NP

            @pl.when(cl < N_CHUNKS)
            def _():
                @pl.when(cl >= NX)
                def _():
                    pltpu.make_async_copy(
                        x_v.at[sx_l], out_hbm.at[rows(cl - NX)],
                        out_sem.at[sx_l]).wait()
                issue_loads(cl, sx_l, sp_l)

            pltpu.make_async_copy(
                pe_hbm.at[idx_ref(c)], pe_v.at[sp], gat_sem.at[sp]).wait()
            pltpu.make_async_copy(
                x_hbm.at[rows(c)], x_v.at[sx], xin_sem.at[sx]).wait()

            n_sl = D_MODEL // LANES

            @plsc.parallel_loop(0, N_SLICES, 1, unroll=8)
            def _(i):
                t = i // n_sl
                j = i - t * n_sl
                d = pl.ds(j * LANES, LANES)
                plsc.addupdate(x_v.at[sx, t, d], pe_v[sp, t, d])

            pltpu.async_copy(x_v.at[sx], out_hbm.at[rows(c)], out_sem.at[sx])
        return 0

    lax.fori_loop(0, N_SUPER, super_step, 0)

    for b in range(NX):
        c = N_CHUNKS - NX + b
        pltpu.make_async_copy(
            x_v.at[b], out_hbm.at[rows(c)], out_sem.at[b]).wait()


@jax.jit
def _pe_add(x2d, pos1d, pe):
    mesh = plsc.VectorSubcoreMesh(core_axis_name="c", subcore_axis_name="s")
    kern = functools.partial(
        pl.kernel,
        mesh=mesh,
        out_type=jax.ShapeDtypeStruct((B_TOTAL, D_MODEL), jnp.float32),
        scratch_types=[
            pltpu.VMEM((B_PER_W,), jnp.int32),
            pltpu.VMEM((NP, CHUNK, D_MODEL), jnp.float32),
            pltpu.VMEM((NX, CHUNK, D_MODEL), jnp.float32),
            pltpu.SemaphoreType.DMA((NP,)),
            pltpu.SemaphoreType.DMA((NX,)),
            pltpu.SemaphoreType.DMA((NX,)),
        ],
    )(_sc_body)
    return kern(x2d, pos1d, pe)


def kernel(x, position, pe):
    b, s, d = x.shape
    x2d = x.reshape(b * s, d)
    pos1d = position.reshape(b * s).astype(jnp.int32)
    out = _pe_add(x2d, pos1d, pe)
    return out.reshape(b, s, d)
